# baseline (speedup 1.0000x reference)
import functools

import jax
import jax.numpy as jnp
from jax import lax
from jax.experimental import pallas as pl
from jax.experimental.pallas import tpu as pltpu
from jax.experimental.pallas import tpu_sc as plsc

N = 10000
E = 320000
D = 128

NC = 2
NS = 16
NW = NC * NS

N_PAD = 10240
ZB = N_PAD // NS
BLK = N_PAD // 8
TRASH = N

CH = 128
C = 80
HB = C // 2
E_PAD = NW * C * CH

_mesh = plsc.VectorSubcoreMesh(
    core_axis_name="c", subcore_axis_name="s", num_cores=NC, num_subcores=NS)


@functools.partial(
    pl.kernel, mesh=_mesh,
    out_type=jax.ShapeDtypeStruct((NC * N_PAD,), jnp.float32),
    scratch_types=[
        pltpu.VMEM((C, CH), jnp.int32),
        pltpu.VMEM((CH,), jnp.float32),
        pltpu.VMEM_SHARED((N_PAD,), jnp.float32),
    ],
)
def _deg_kernel(col_hbm, zeros_hbm, deg_out, col_v, ones_v, deg_sp):
    cid = lax.axis_index("c")
    sid = lax.axis_index("s")
    wid = cid * NS + sid

    one = jnp.ones((16,), jnp.float32)

    def fill_ones(i, _):
        ones_v[pl.ds(i * 16, 16)] = one
        return 0
    lax.fori_loop(0, CH // 16, fill_ones, 0)

    pltpu.sync_copy(zeros_hbm, deg_sp.at[pl.ds(sid * ZB, ZB)])
    plsc.subcore_barrier()

    pltpu.sync_copy(col_hbm.at[wid], col_v)

    def body(j, _):
        pltpu.sync_copy(ones_v, deg_sp.at[col_v.at[j]], add=True)
        return 0
    lax.fori_loop(0, C, body, 0)

    plsc.subcore_barrier()
    pltpu.sync_copy(deg_sp.at[pl.ds(sid * ZB, ZB)],
                    deg_out.at[pl.ds(cid * N_PAD + sid * ZB, ZB)])


@functools.partial(
    pl.kernel, mesh=_mesh,
    out_type=jax.ShapeDtypeStruct((NC, N_PAD, D), jnp.float32),
    scratch_types=[
        pltpu.VMEM((C, CH), jnp.int32),
        pltpu.VMEM((C, CH), jnp.int32),
        pltpu.VMEM((CH, D), jnp.float32),
        pltpu.VMEM_SHARED((N_PAD, D), jnp.float32),
        pltpu.SemaphoreType.DMA,
    ],
)
def _agg_kernel(y_hbm, row_hbm, col_hbm, zrows_hbm, s_out,
                row_v, col_v, buf, s_sp, sem):
    cid = lax.axis_index("c")
    sid = lax.axis_index("s")
    wid = cid * NS + sid

    pltpu.sync_copy(zrows_hbm, s_sp.at[pl.ds(sid * ZB, ZB)])
    plsc.subcore_barrier()

    pltpu.sync_copy(row_hbm.at[wid], row_v)
    pltpu.sync_copy(col_hbm.at[wid], col_v)

    def body(j, _):
        pltpu.async_copy(y_hbm.at[row_v.at[j]], buf, sem).wait()
        pltpu.sync_copy(buf, s_sp.at[col_v.at[j]], add=True)
        return 0
    lax.fori_loop(0, C, body, 0)

    plsc.subcore_barrier()
    pltpu.sync_copy(s_sp.at[pl.ds(sid * ZB, ZB)],
                    s_out.at[cid, pl.ds(sid * ZB, ZB)])


def _transform_body(x_ref, w_ref, dp_ref, y_ref, dis_ref):
    deg = dp_ref[0, :, 0] + dp_ref[1, :, 0] + 1.0
    dis = lax.rsqrt(deg)
    dis_ref[...] = dis[:, None]
    xw = jnp.dot(x_ref[...], w_ref[...], preferred_element_type=jnp.float32)
    y_ref[...] = xw * dis[:, None]


def _finalize_body(sp_ref, y_ref, dis_ref, b_ref, o_ref):
    s = sp_ref[0] + sp_ref[1] + y_ref[...]
    o_ref[...] = jnp.maximum(s * dis_ref[...] + b_ref[0, :], 0.0)


def kernel(x, edge_index, W, b):
    row = edge_index[0].astype(jnp.int32)
    col = edge_index[1].astype(jnp.int32)
    row_c = jnp.concatenate(
        [row, jnp.zeros((E_PAD - E,), jnp.int32)]).reshape(NW, C, CH)
    col_c = jnp.concatenate(
        [col, jnp.full((E_PAD - E,), TRASH, jnp.int32)]).reshape(NW, C, CH)
    x_pad = jnp.pad(x, ((0, N_PAD - N), (0, 0)))
    z_deg = jnp.zeros((ZB,), jnp.float32)
    z_rows = jnp.zeros((ZB, D), jnp.float32)

    deg_p = _deg_kernel(col_c, z_deg)

    y, dis = pl.pallas_call(
        _transform_body,
        grid=(N_PAD // BLK,),
        in_specs=[
            pl.BlockSpec((BLK, D), lambda i: (i, 0)),
            pl.BlockSpec((D, D), lambda i: (0, 0)),
            pl.BlockSpec((NC, BLK, 1), lambda i: (0, i, 0)),
        ],
        out_specs=[
            pl.BlockSpec((BLK, D), lambda i: (i, 0)),
            pl.BlockSpec((BLK, 1), lambda i: (i, 0)),
        ],
        out_shape=[
            jax.ShapeDtypeStruct((N_PAD, D), jnp.float32),
            jax.ShapeDtypeStruct((N_PAD, 1), jnp.float32),
        ],
    )(x_pad, W, deg_p.reshape(NC, N_PAD, 1))

    s_p = _agg_kernel(y, row_c, col_c, z_rows)

    out = pl.pallas_call(
        _finalize_body,
        grid=(N_PAD // BLK,),
        in_specs=[
            pl.BlockSpec((NC, BLK, D), lambda i: (0, i, 0)),
            pl.BlockSpec((BLK, D), lambda i: (i, 0)),
            pl.BlockSpec((BLK, 1), lambda i: (i, 0)),
            pl.BlockSpec((1, D), lambda i: (0, 0)),
        ],
        out_specs=pl.BlockSpec((BLK, D), lambda i: (i, 0)),
        out_shape=jax.ShapeDtypeStruct((N_PAD, D), jnp.float32),
    )(s_p, y, dis, b.reshape(1, D))
    return out[:N]


# speedup vs baseline: 1.1670x; 1.1670x over previous
import functools

import jax
import jax.numpy as jnp
from jax import lax
from jax.experimental import pallas as pl
from jax.experimental.pallas import tpu as pltpu
from jax.experimental.pallas import tpu_sc as plsc

N = 10000
E = 320000
D = 128

NC = 2
NS = 16
NW = NC * NS

N_PAD = 10240
ZB = N_PAD // NS
BLK = N_PAD // 8
TRASH = N

CH = 128
C = 80
HB = C // 2
E_PAD = NW * C * CH

_mesh = plsc.VectorSubcoreMesh(
    core_axis_name="c", subcore_axis_name="s", num_cores=NC, num_subcores=NS)


@functools.partial(
    pl.kernel, mesh=_mesh,
    out_type=jax.ShapeDtypeStruct((NC * N_PAD,), jnp.float32),
    scratch_types=[
        pltpu.VMEM((C, CH), jnp.int32),
        pltpu.VMEM((CH,), jnp.float32),
        pltpu.VMEM_SHARED((N_PAD,), jnp.float32),
    ],
)
def _deg_kernel(col_hbm, zeros_hbm, deg_out, col_v, ones_v, deg_sp):
    cid = lax.axis_index("c")
    sid = lax.axis_index("s")
    wid = cid * NS + sid

    one = jnp.ones((16,), jnp.float32)

    def fill_ones(i, _):
        ones_v[pl.ds(i * 16, 16)] = one
        return 0
    lax.fori_loop(0, CH // 16, fill_ones, 0)

    pltpu.sync_copy(zeros_hbm, deg_sp.at[pl.ds(sid * ZB, ZB)])
    plsc.subcore_barrier()

    pltpu.sync_copy(col_hbm.at[wid], col_v)

    def body(j, _):
        pltpu.sync_copy(ones_v, deg_sp.at[col_v.at[j]], add=True)
        return 0
    lax.fori_loop(0, C, body, 0)

    plsc.subcore_barrier()
    pltpu.sync_copy(deg_sp.at[pl.ds(sid * ZB, ZB)],
                    deg_out.at[pl.ds(cid * N_PAD + sid * ZB, ZB)])


@functools.partial(
    pl.kernel, mesh=_mesh,
    out_type=jax.ShapeDtypeStruct((NC, N_PAD, D), jnp.float32),
    scratch_types=[
        pltpu.VMEM((C, CH), jnp.int32),
        pltpu.VMEM((C, CH), jnp.int32),
        pltpu.VMEM((CH, D), jnp.float32),
        pltpu.VMEM_SHARED((N_PAD, D), jnp.float32),
        pltpu.SemaphoreType.DMA,
    ],
)
def _agg_kernel(y_hbm, row_hbm, col_hbm, zrows_hbm, s_out,
                row_v, col_v, buf, s_sp, sem):
    cid = lax.axis_index("c")
    sid = lax.axis_index("s")
    wid = cid * NS + sid

    pltpu.sync_copy(zrows_hbm, s_sp.at[pl.ds(sid * ZB, ZB)])
    plsc.subcore_barrier()

    pltpu.sync_copy(row_hbm.at[wid], row_v)
    pltpu.sync_copy(col_hbm.at[wid], col_v)

    def body(j, _):
        pltpu.async_copy(y_hbm.at[row_v.at[j]], buf, sem).wait()
        pltpu.sync_copy(buf, s_sp.at[col_v.at[j]], add=True)
        return 0
    lax.fori_loop(0, C, body, 0)

    plsc.subcore_barrier()
    pltpu.sync_copy(s_sp.at[pl.ds(sid * ZB, ZB)],
                    s_out.at[cid, pl.ds(sid * ZB, ZB)])


def _transform_body(x_ref, w_ref, dp_ref, y_ref, dis_ref):
    deg = dp_ref[0, :] + dp_ref[1, :] + 1.0
    dis = lax.rsqrt(deg)
    dis_ref[...] = dis[None, :]
    xw = jnp.dot(x_ref[...], w_ref[...], preferred_element_type=jnp.float32)
    y_ref[...] = xw * dis[:, None]


def _finalize_body(sp_ref, y_ref, dis_ref, b_ref, o_ref):
    s = sp_ref[0] + sp_ref[1] + y_ref[...]
    o_ref[...] = jnp.maximum(s * dis_ref[0, :][:, None] + b_ref[0, :], 0.0)


def kernel(x, edge_index, W, b):
    row = edge_index[0].astype(jnp.int32)
    col = edge_index[1].astype(jnp.int32)
    row_c = jnp.concatenate(
        [row, jnp.zeros((E_PAD - E,), jnp.int32)]).reshape(NW, C, CH)
    col_c = jnp.concatenate(
        [col, jnp.full((E_PAD - E,), TRASH, jnp.int32)]).reshape(NW, C, CH)
    x_pad = jnp.pad(x, ((0, N_PAD - N), (0, 0)))
    z_deg = jnp.zeros((ZB,), jnp.float32)
    z_rows = jnp.zeros((ZB, D), jnp.float32)

    deg_p = _deg_kernel(col_c, z_deg)

    y, dis = pl.pallas_call(
        _transform_body,
        grid=(N_PAD // BLK,),
        in_specs=[
            pl.BlockSpec((BLK, D), lambda i: (i, 0)),
            pl.BlockSpec((D, D), lambda i: (0, 0)),
            pl.BlockSpec((NC, BLK), lambda i: (0, i)),
        ],
        out_specs=[
            pl.BlockSpec((BLK, D), lambda i: (i, 0)),
            pl.BlockSpec((1, BLK), lambda i: (0, i)),
        ],
        out_shape=[
            jax.ShapeDtypeStruct((N_PAD, D), jnp.float32),
            jax.ShapeDtypeStruct((1, N_PAD), jnp.float32),
        ],
    )(x_pad, W, deg_p.reshape(NC, N_PAD))

    s_p = _agg_kernel(y, row_c, col_c, z_rows)

    out = pl.pallas_call(
        _finalize_body,
        grid=(N_PAD // BLK,),
        in_specs=[
            pl.BlockSpec((NC, BLK, D), lambda i: (0, i, 0)),
            pl.BlockSpec((BLK, D), lambda i: (i, 0)),
            pl.BlockSpec((1, BLK), lambda i: (0, i)),
            pl.BlockSpec((1, D), lambda i: (0, 0)),
        ],
        out_specs=pl.BlockSpec((BLK, D), lambda i: (i, 0)),
        out_shape=jax.ShapeDtypeStruct((N_PAD, D), jnp.float32),
    )(s_p, y, dis, b.reshape(1, D))
    return out[:N]
